# Initial kernel scaffold; baseline (speedup 1.0000x reference)
import jax
import jax.numpy as jnp
from jax.experimental import pallas as pl


def kernel(x, edge_index, batch, edge_attr, params):
    raise NotImplementedError("write your pallas kernel here")


# speedup vs baseline: 9.2684x; 9.2684x over previous
import functools

import jax
import jax.numpy as jnp
from jax import lax
from jax.experimental import pallas as pl
from jax.experimental.pallas import tpu as pltpu
from jax.experimental.pallas import tpu_sc as plsc

N = 10000
D = 128
NP = 10112
E = 320000
NCORES = 2
NSUB = 16
NTILES = NCORES * NSUB
C = 128
EPT = 10240
EP = NTILES * EPT
NCH = EPT // C
RPS = NP // NSUB
ZR = 80

_MESH = plsc.VectorSubcoreMesh(core_axis_name="c", subcore_axis_name="s")


def _prep_body(ee_ref, we_ref, afe_ref, sel_ref, oh_ref, et_ref, ltc_ref):
    for i in range(4):
        ehh = jnp.dot(ee_ref[...], we_ref[i], preferred_element_type=jnp.float32)
        tab = jnp.dot(ehh * afe_ref[i], sel_ref[i],
                      preferred_element_type=jnp.float32)
        et_ref[i] = tab
        if i < 2:
            ltc_ref[i] = tab + oh_ref[...]


def _dense_body(x_ref, w_ref, afs_ref, afd_ref, sel_ref,
                h_ref, as_ref, ad_ref):
    h = jnp.dot(x_ref[...], w_ref[...], preferred_element_type=jnp.float32)
    h_ref[...] = h
    as_ref[...] = jnp.dot(h * afs_ref[...], sel_ref[...],
                          preferred_element_type=jnp.float32)
    ad_ref[...] = jnp.dot(h * afd_ref[...], sel_ref[...],
                          preferred_element_type=jnp.float32)


def _mid_loop_body(d0_ref, d1_ref, l0_ref, l1_ref, as_ref, ad_ref,
                   rden_ref, cl_ref):
    den = d0_ref[...] + d1_ref[...]
    ltc = l0_ref[...] + l1_ref[...]
    cnt = ltc[:, 8:9]
    la = as_ref[...] + ad_ref[...] + ltc / jnp.maximum(cnt, 1.0)
    lex = jnp.exp(jnp.where(la > 0, la, 0.2 * la))
    rden = 1.0 / jnp.maximum(den + lex, 1e-16)
    rden_ref[...] = rden
    cl_ref[...] = lex * rden


def _mid_plain_body(d0_ref, d1_ref, rden_ref):
    rden_ref[...] = 1.0 / jnp.maximum(d0_ref[...] + d1_ref[...], 1e-16)


def _ep_body(has_cl, has_res, has_dense, *refs):
    i = 0
    o0_ref = refs[i]; i += 1
    o1_ref = refs[i]; i += 1
    b_ref = refs[i]; i += 1
    g = o0_ref[...] + o1_ref[...] + b_ref[...]
    if has_cl:
        h_ref = refs[i]; i += 1
        cl_ref = refs[i]; i += 1
        ex_ref = refs[i]; i += 1
        g = g + jnp.dot(cl_ref[...], ex_ref[...],
                        preferred_element_type=jnp.float32) * h_ref[...]
    g = jnp.where(g > 0, g, 0.0)
    if has_res:
        r_ref = refs[i]; i += 1
        g = g + r_ref[...]
    if has_dense:
        w_ref = refs[i]; i += 1
        afs_ref = refs[i]; i += 1
        afd_ref = refs[i]; i += 1
        sel_ref = refs[i]; i += 1
        hn_ref = refs[i]; i += 1
        hf_ref = refs[i]; i += 1
        as_ref = refs[i]; i += 1
        ad_ref = refs[i]; i += 1
        hn_ref[...] = g
        hf = jnp.dot(g, w_ref[...], preferred_element_type=jnp.float32)
        hf_ref[...] = hf
        as_ref[...] = jnp.dot(hf * afs_ref[...], sel_ref[...],
                              preferred_element_type=jnp.float32)
        ad_ref[...] = jnp.dot(hf * afd_ref[...], sel_ref[...],
                              preferred_element_type=jnp.float32)
    else:
        hn_ref = refs[i]; i += 1
        hn_ref[...] = g


_GB = 10
_BN = N // _GB


def _row_spec(w):
    return pl.BlockSpec((_BN, w), lambda i: (i, 0))


def _full_spec(shape):
    return pl.BlockSpec(shape, lambda i: tuple(0 for _ in shape))


def _tc_dense(x, w, afs, afd, sel):
    return pl.pallas_call(
        _dense_body,
        grid=(_GB,),
        in_specs=[_row_spec(D), _full_spec((D, D)), _full_spec((1, D)),
                  _full_spec((1, D)), _full_spec((D, 16))],
        out_specs=[_row_spec(D), _row_spec(16), _row_spec(16)],
        out_shape=[jax.ShapeDtypeStruct((N, D), jnp.float32),
                   jax.ShapeDtypeStruct((N, 16), jnp.float32),
                   jax.ShapeDtypeStruct((N, 16), jnp.float32)],
    )(x, w, afs, afd, sel)


def _tc_mid_loop(d0, d1, l0, l1, a_s, a_d):
    return pl.pallas_call(
        _mid_loop_body,
        grid=(_GB,),
        in_specs=[_row_spec(16)] * 6,
        out_specs=[_row_spec(16)] * 2,
        out_shape=[jax.ShapeDtypeStruct((N, 16), jnp.float32),
                   jax.ShapeDtypeStruct((N, 16), jnp.float32)],
    )(d0, d1, l0, l1, a_s, a_d)


def _tc_mid_plain(d0, d1):
    return pl.pallas_call(
        _mid_plain_body,
        grid=(_GB,),
        in_specs=[_row_spec(16)] * 2,
        out_specs=_row_spec(16),
        out_shape=jax.ShapeDtypeStruct((N, 16), jnp.float32),
    )(d0, d1)


def _tc_ep(o0, o1, b, cl_args, res, dense_args):
    has_cl = cl_args is not None
    has_res = res is not None
    has_dense = dense_args is not None
    ins = [o0, o1, b]
    in_specs = [_row_spec(D), _row_spec(D), _full_spec((1, D))]
    if has_cl:
        hprev, cl, ex = cl_args
        ins += [hprev, cl, ex]
        in_specs += [_row_spec(D), _row_spec(16), _full_spec((16, D))]
    if has_res:
        ins.append(res)
        in_specs.append(_row_spec(D))
    if has_dense:
        w, afs, afd, sel = dense_args
        ins += [w, afs, afd, sel]
        in_specs += [_full_spec((D, D)), _full_spec((1, D)),
                     _full_spec((1, D)), _full_spec((D, 16))]
        out_specs = [_row_spec(D), _row_spec(D), _row_spec(16), _row_spec(16)]
        out_shape = [jax.ShapeDtypeStruct((N, D), jnp.float32),
                     jax.ShapeDtypeStruct((N, D), jnp.float32),
                     jax.ShapeDtypeStruct((N, 16), jnp.float32),
                     jax.ShapeDtypeStruct((N, 16), jnp.float32)]
    else:
        out_specs = _row_spec(D)
        out_shape = jax.ShapeDtypeStruct((N, D), jnp.float32)
    return pl.pallas_call(
        functools.partial(_ep_body, has_cl, has_res, has_dense),
        grid=(_GB,),
        in_specs=in_specs,
        out_specs=out_specs,
        out_shape=out_shape,
    )(*ins)


def _tc_prep(ee, we4, afe4, sel4, oh):
    return pl.pallas_call(
        _prep_body,
        out_shape=[jax.ShapeDtypeStruct((4, 8, 16), jnp.float32),
                   jax.ShapeDtypeStruct((2, 8, 16), jnp.float32)],
    )(ee, we4, afe4, sel4, oh)


def _mask_dst(mode, t, d):
    if mode == 0:
        m = t <= 1
    elif mode == 1:
        m = t == 2
    else:
        m = t == 1
    return jnp.where(m, d, N)


def _sc_pass_a(mode, has_ltc):
    scratch = [
        pltpu.VMEM((C,), jnp.int32),
        pltpu.VMEM((C,), jnp.int32),
        pltpu.VMEM((C,), jnp.int32),
        pltpu.VMEM((C,), jnp.int32),
        pltpu.VMEM((C, 16), jnp.float32),
        pltpu.VMEM((C, 16), jnp.float32),
        pltpu.VMEM((RPS, 16), jnp.float32),
        pltpu.SemaphoreType.DMA,
        pltpu.VMEM_SHARED((NP, 16), jnp.float32),
    ]
    out_type = [jax.ShapeDtypeStruct((EP, 16), jnp.float32),
                jax.ShapeDtypeStruct((NCORES, NP, 16), jnp.float32)]
    if has_ltc:
        scratch.append(pltpu.VMEM_SHARED((NP, 16), jnp.float32))
        out_type.append(jax.ShapeDtypeStruct((NCORES, NP, 16), jnp.float32))

    def body(*refs):
        if has_ltc:
            (sarr, darr, tarr, as16, ad16, et16, ltc16,
             ex_o, den_o, ltc_o,
             sb, db, tb, mdb, acc, ltb, zb, sem, den_acc, ltc_acc) = refs
        else:
            (sarr, darr, tarr, as16, ad16, et16,
             ex_o, den_o,
             sb, db, tb, mdb, acc, ltb, zb, sem, den_acc) = refs
        c = lax.axis_index("c")
        s = lax.axis_index("s")

        @pl.loop(0, RPS)
        def _zero(i):
            zb[i] = jnp.zeros((16,), jnp.float32)

        pltpu.sync_copy(zb, den_acc.at[pl.ds(s * RPS, RPS)])
        if has_ltc:
            pltpu.sync_copy(zb, ltc_acc.at[pl.ds(s * RPS, RPS)])
        plsc.subcore_barrier()

        base0 = (c * NSUB + s) * EPT

        @pl.loop(0, NCH)
        def _chunk(g):
            base = base0 + g * C
            pltpu.sync_copy(sarr.at[pl.ds(base, C)], sb)
            pltpu.sync_copy(darr.at[pl.ds(base, C)], db)
            pltpu.sync_copy(tarr.at[pl.ds(base, C)], tb)

            @pl.loop(0, C // 16)
            def _mask(j):
                t = tb[pl.ds(j * 16, 16)]
                d = db[pl.ds(j * 16, 16)]
                mdb[pl.ds(j * 16, 16)] = _mask_dst(mode, t, d)

            pltpu.async_copy(et16.at[tb], acc, sem).wait()
            pltpu.async_copy(as16.at[sb], acc, sem, add=True).wait()
            pltpu.async_copy(ad16.at[mdb], acc, sem, add=True).wait()

            @pl.loop(0, C)
            def _exp(e):
                v = acc[e]
                v = jnp.exp(jnp.where(v > 0, v, 0.2 * v))
                acc[e] = v

            pltpu.sync_copy(acc, ex_o.at[pl.ds(base, C)])
            pltpu.sync_copy(acc, den_acc.at[mdb], add=True)
            if has_ltc:
                pltpu.async_copy(ltc16.at[tb], ltb, sem).wait()
                pltpu.sync_copy(ltb, ltc_acc.at[mdb], add=True)

        plsc.subcore_barrier()
        pltpu.sync_copy(den_acc.at[pl.ds(s * RPS, RPS)],
                        den_o.at[c, pl.ds(s * RPS, RPS)])
        if has_ltc:
            pltpu.sync_copy(ltc_acc.at[pl.ds(s * RPS, RPS)],
                            ltc_o.at[c, pl.ds(s * RPS, RPS)])

    return pl.kernel(body, out_type=tuple(out_type), mesh=_MESH,
                     scratch_types=tuple(scratch),
                     compiler_params=pltpu.CompilerParams(
                         use_tc_tiling_on_sc=False))


def _sc_pass_b(mode):
    scratch = [
        pltpu.VMEM((C,), jnp.int32),
        pltpu.VMEM((C,), jnp.int32),
        pltpu.VMEM((C,), jnp.int32),
        pltpu.VMEM((C,), jnp.int32),
        pltpu.VMEM((C, 16), jnp.float32),
        pltpu.VMEM((C, 16), jnp.float32),
        pltpu.VMEM((C, D), jnp.float32),
        pltpu.VMEM((ZR, D), jnp.float32),
        pltpu.SemaphoreType.DMA,
        pltpu.VMEM_SHARED((NP, D), jnp.float32),
    ]

    def body(sarr, darr, tarr, ex_i, rden, h,
             out_o,
             sb, db, tb, mdb, exb, rdb, hb, zb, sem, out_acc):
        c = lax.axis_index("c")
        s = lax.axis_index("s")

        @pl.loop(0, ZR)
        def _zero(i):
            for j in range(D // 16):
                zb[i, pl.ds(j * 16, 16)] = jnp.zeros((16,), jnp.float32)

        for k in range(RPS // ZR):
            pltpu.sync_copy(zb, out_acc.at[pl.ds(s * RPS + k * ZR, ZR)])
        rem = RPS % ZR
        if rem:
            pltpu.sync_copy(zb.at[pl.ds(0, rem)],
                            out_acc.at[pl.ds(s * RPS + (RPS // ZR) * ZR, rem)])
        plsc.subcore_barrier()

        base0 = (c * NSUB + s) * EPT

        @pl.loop(0, NCH)
        def _chunk(g):
            base = base0 + g * C
            pltpu.sync_copy(sarr.at[pl.ds(base, C)], sb)
            pltpu.sync_copy(darr.at[pl.ds(base, C)], db)
            pltpu.sync_copy(tarr.at[pl.ds(base, C)], tb)

            @pl.loop(0, C // 16)
            def _mask(j):
                t = tb[pl.ds(j * 16, 16)]
                d = db[pl.ds(j * 16, 16)]
                mdb[pl.ds(j * 16, 16)] = _mask_dst(mode, t, d)

            pltpu.sync_copy(ex_i.at[pl.ds(base, C)], exb)
            pltpu.async_copy(rden.at[mdb], rdb, sem).wait()
            pltpu.async_copy(h.at[sb], hb, sem).wait()

            @pl.loop(0, C)
            def _scale(e):
                cfv = exb[e] * rdb[e]
                for j in range(D // 16):
                    hb[e, pl.ds(j * 16, 16)] = (
                        hb[e, pl.ds(j * 16, 16)] * cfv[j])

            pltpu.sync_copy(hb, out_acc.at[mdb], add=True)

        plsc.subcore_barrier()
        pltpu.sync_copy(out_acc.at[pl.ds(s * RPS, RPS)],
                        out_o.at[c, pl.ds(s * RPS, RPS)])

    return pl.kernel(
        body,
        out_type=jax.ShapeDtypeStruct((NCORES, NP, D), jnp.float32),
        mesh=_MESH,
        scratch_types=tuple(scratch),
        compiler_params=pltpu.CompilerParams(use_tc_tiling_on_sc=False))


def _sel8():
    r = jnp.arange(D)[:, None]
    c_ = jnp.arange(16)[None, :]
    return jnp.where((r // 16) == c_, 1.0, 0.0).astype(jnp.float32)


def _expand16():
    r = jnp.arange(16)[:, None]
    c_ = jnp.arange(D)[None, :]
    return jnp.where(r == (c_ // 16), 1.0, 0.0).astype(jnp.float32)


def _pad_nodes(a):
    return jnp.pad(a, ((0, NP - N), (0, 0)))


def kernel(x, edge_index, batch, edge_attr, params):
    p1, p2, p3, p4 = (params['conv1'], params['conv2'],
                      params['conv3'], params['conv4'])
    sel8 = _sel8()
    sel1 = jnp.ones((D, 16), jnp.float32)
    expand = _expand16()

    pad = EP - E
    src0 = jnp.concatenate([edge_index[0].astype(jnp.int32),
                            jnp.zeros((pad,), jnp.int32)])
    dst0 = jnp.concatenate([edge_index[1].astype(jnp.int32),
                            jnp.zeros((pad,), jnp.int32)])
    typ = jnp.concatenate([edge_attr.astype(jnp.int32),
                           jnp.full((pad,), 7, jnp.int32)])

    we4 = jnp.stack([p['W_e'] for p in (p1, p2, p3, p4)])
    afe4 = jnp.stack([p['att_e'].reshape(1, -1) if p['att_e'].size == D
                      else p['att_e'].reshape(1, -1)
                      for p in (p1, p2, p3, p4)])
    sel4 = jnp.stack([sel8, sel8, sel1, sel1])
    oh = (jnp.arange(16) == 8).astype(jnp.float32).reshape(1, 16)
    et4, ltc2 = _tc_prep(params['edge_emb'], we4, afe4, sel4, oh)

    sc_a_loop = _sc_pass_a(0, True)
    sc_a3 = _sc_pass_a(1, False)
    sc_a4 = _sc_pass_a(2, False)
    sc_b = [_sc_pass_b(m) for m in (0, 0, 1, 2)]

    def gat_loop(li, hin, w, afs, afd, p, et16, ltc16, res, next_dense):
        h, a_s, a_d = hin
        ex, den_r, ltc_r = sc_a_loop(src0, dst0, typ,
                                     _pad_nodes(a_s), _pad_nodes(a_d),
                                     et16, ltc16)
        rden, cl = _tc_mid_loop(den_r[0, :N], den_r[1, :N],
                                ltc_r[0, :N], ltc_r[1, :N], a_s, a_d)
        outp = sc_b[li](src0, dst0, typ, ex, _pad_nodes(rden), h)
        b = p['b'].reshape(1, D)
        return _tc_ep(outp[0, :N], outp[1, :N], b,
                      (h, cl, expand), res, next_dense)

    def gat_plain(li, hin, p, sc_a, sarr, darr, res, next_dense):
        h, a_s, a_d = hin
        ex, den_r = sc_a(sarr, darr, typ,
                         _pad_nodes(a_s), _pad_nodes(a_d), et4[li])
        rden = _tc_mid_plain(den_r[0, :N], den_r[1, :N])
        outp = sc_b[li](sarr, darr, typ, ex, _pad_nodes(rden), h)
        b = p['b'].reshape(1, D)
        return _tc_ep(outp[0, :N], outp[1, :N], b, None, res, next_dense)

    def flat(p):
        return (p['W'], p['att_src'].reshape(1, D), p['att_dst'].reshape(1, D))

    w1, afs1, afd1 = flat(p1)
    w2, afs2, afd2 = flat(p2)
    w3, afs3, afd3 = flat(p3)
    w4, afs4, afd4 = flat(p4)

    h1, as1, ad1 = _tc_dense(x, w1, afs1, afd1, sel8)

    hA, h2f, as2, ad2 = gat_loop(0, (h1, as1, ad1), w1, afs1, afd1, p1,
                                 et4[0], ltc2[0], None,
                                 (w2, afs2, afd2, sel8))
    hB, h3f, as3, ad3 = gat_loop(1, (h2f, as2, ad2), w2, afs2, afd2, p2,
                                 et4[1], ltc2[1], hA,
                                 (w3, afs3, afd3, sel1))
    hC, h4f, as4, ad4 = gat_plain(2, (h3f, as3, ad3), p3, sc_a3,
                                  src0, dst0, hB,
                                  (w4, afs4, afd4, sel1))
    hD = gat_plain(3, (h4f, as4, ad4), p4, sc_a4,
                   dst0, src0, hC, None)
    return (hD, batch)


# speedup vs baseline: 9.4648x; 1.0212x over previous
import functools

import jax
import jax.numpy as jnp
from jax import lax
from jax.experimental import pallas as pl
from jax.experimental.pallas import tpu as pltpu
from jax.experimental.pallas import tpu_sc as plsc

N = 10000
D = 128
NP = 10112
E = 320000
NCORES = 2
NSUB = 16
NTILES = NCORES * NSUB
C = 128
EPT = 10240
EP = NTILES * EPT
NCH = EPT // C
RPS = NP // NSUB
ZR = 80

_MESH = plsc.VectorSubcoreMesh(core_axis_name="c", subcore_axis_name="s")


def _prep_body(ee_ref, we_ref, afe_ref, sel_ref, oh_ref, et_ref, ltc_ref):
    for i in range(4):
        ehh = jnp.dot(ee_ref[...], we_ref[i], preferred_element_type=jnp.float32)
        tab = jnp.dot(ehh * afe_ref[i], sel_ref[i],
                      preferred_element_type=jnp.float32)
        et_ref[i] = tab
        if i < 2:
            ltc_ref[i] = tab + oh_ref[...]


def _dense_body(x_ref, w_ref, afs_ref, afd_ref, sel_ref,
                h_ref, as_ref, ad_ref):
    h = jnp.dot(x_ref[...], w_ref[...], preferred_element_type=jnp.float32)
    h_ref[...] = h
    as_ref[...] = jnp.dot(h * afs_ref[...], sel_ref[...],
                          preferred_element_type=jnp.float32)
    ad_ref[...] = jnp.dot(h * afd_ref[...], sel_ref[...],
                          preferred_element_type=jnp.float32)


def _mid_loop_body(d0_ref, d1_ref, l0_ref, l1_ref, as_ref, ad_ref,
                   rden_ref, cl_ref):
    den = d0_ref[...] + d1_ref[...]
    ltc = l0_ref[...] + l1_ref[...]
    cnt = ltc[:, 8:9]
    la = as_ref[...] + ad_ref[...] + ltc / jnp.maximum(cnt, 1.0)
    lex = jnp.exp(jnp.where(la > 0, la, 0.2 * la))
    rden = 1.0 / jnp.maximum(den + lex, 1e-16)
    rden_ref[...] = rden
    cl_ref[...] = lex * rden


def _mid_plain_body(d0_ref, d1_ref, rden_ref):
    rden_ref[...] = 1.0 / jnp.maximum(d0_ref[...] + d1_ref[...], 1e-16)


def _ep_body(has_cl, has_res, has_dense, *refs):
    i = 0
    o0_ref = refs[i]; i += 1
    o1_ref = refs[i]; i += 1
    b_ref = refs[i]; i += 1
    g = o0_ref[...] + o1_ref[...] + b_ref[...]
    if has_cl:
        h_ref = refs[i]; i += 1
        cl_ref = refs[i]; i += 1
        ex_ref = refs[i]; i += 1
        g = g + jnp.dot(cl_ref[...], ex_ref[...],
                        preferred_element_type=jnp.float32) * h_ref[...]
    g = jnp.where(g > 0, g, 0.0)
    if has_res:
        r_ref = refs[i]; i += 1
        g = g + r_ref[...]
    if has_dense:
        w_ref = refs[i]; i += 1
        afs_ref = refs[i]; i += 1
        afd_ref = refs[i]; i += 1
        sel_ref = refs[i]; i += 1
        hn_ref = refs[i]; i += 1
        hf_ref = refs[i]; i += 1
        as_ref = refs[i]; i += 1
        ad_ref = refs[i]; i += 1
        hn_ref[...] = g
        hf = jnp.dot(g, w_ref[...], preferred_element_type=jnp.float32)
        hf_ref[...] = hf
        as_ref[...] = jnp.dot(hf * afs_ref[...], sel_ref[...],
                              preferred_element_type=jnp.float32)
        ad_ref[...] = jnp.dot(hf * afd_ref[...], sel_ref[...],
                              preferred_element_type=jnp.float32)
    else:
        hn_ref = refs[i]; i += 1
        hn_ref[...] = g


_GB = 10
_BN = N // _GB


def _row_spec(w):
    return pl.BlockSpec((_BN, w), lambda i: (i, 0))


def _full_spec(shape):
    return pl.BlockSpec(shape, lambda i: tuple(0 for _ in shape))


def _tc_dense(x, w, afs, afd, sel):
    return pl.pallas_call(
        _dense_body,
        grid=(_GB,),
        in_specs=[_row_spec(D), _full_spec((D, D)), _full_spec((1, D)),
                  _full_spec((1, D)), _full_spec((D, 16))],
        out_specs=[_row_spec(D), _row_spec(16), _row_spec(16)],
        out_shape=[jax.ShapeDtypeStruct((N, D), jnp.float32),
                   jax.ShapeDtypeStruct((N, 16), jnp.float32),
                   jax.ShapeDtypeStruct((N, 16), jnp.float32)],
    )(x, w, afs, afd, sel)


def _tc_mid_loop(d0, d1, l0, l1, a_s, a_d):
    return pl.pallas_call(
        _mid_loop_body,
        grid=(_GB,),
        in_specs=[_row_spec(16)] * 6,
        out_specs=[_row_spec(16)] * 2,
        out_shape=[jax.ShapeDtypeStruct((N, 16), jnp.float32),
                   jax.ShapeDtypeStruct((N, 16), jnp.float32)],
    )(d0, d1, l0, l1, a_s, a_d)


def _tc_mid_plain(d0, d1):
    return pl.pallas_call(
        _mid_plain_body,
        grid=(_GB,),
        in_specs=[_row_spec(16)] * 2,
        out_specs=_row_spec(16),
        out_shape=jax.ShapeDtypeStruct((N, 16), jnp.float32),
    )(d0, d1)


def _tc_ep(o0, o1, b, cl_args, res, dense_args):
    has_cl = cl_args is not None
    has_res = res is not None
    has_dense = dense_args is not None
    ins = [o0, o1, b]
    in_specs = [_row_spec(D), _row_spec(D), _full_spec((1, D))]
    if has_cl:
        hprev, cl, ex = cl_args
        ins += [hprev, cl, ex]
        in_specs += [_row_spec(D), _row_spec(16), _full_spec((16, D))]
    if has_res:
        ins.append(res)
        in_specs.append(_row_spec(D))
    if has_dense:
        w, afs, afd, sel = dense_args
        ins += [w, afs, afd, sel]
        in_specs += [_full_spec((D, D)), _full_spec((1, D)),
                     _full_spec((1, D)), _full_spec((D, 16))]
        out_specs = [_row_spec(D), _row_spec(D), _row_spec(16), _row_spec(16)]
        out_shape = [jax.ShapeDtypeStruct((N, D), jnp.float32),
                     jax.ShapeDtypeStruct((N, D), jnp.float32),
                     jax.ShapeDtypeStruct((N, 16), jnp.float32),
                     jax.ShapeDtypeStruct((N, 16), jnp.float32)]
    else:
        out_specs = _row_spec(D)
        out_shape = jax.ShapeDtypeStruct((N, D), jnp.float32)
    return pl.pallas_call(
        functools.partial(_ep_body, has_cl, has_res, has_dense),
        grid=(_GB,),
        in_specs=in_specs,
        out_specs=out_specs,
        out_shape=out_shape,
    )(*ins)


def _tc_prep(ee, we4, afe4, sel4, oh):
    return pl.pallas_call(
        _prep_body,
        out_shape=[jax.ShapeDtypeStruct((4, 8, 16), jnp.float32),
                   jax.ShapeDtypeStruct((2, 8, 16), jnp.float32)],
    )(ee, we4, afe4, sel4, oh)


def _mask_dst(mode, t, d):
    if mode == 0:
        m = t <= 1
    elif mode == 1:
        m = t == 2
    else:
        m = t == 1
    return jnp.where(m, d, N)


def _sc_pass_a(mode, has_ltc):
    NSLOT = 2
    nbuf = 5 if has_ltc else 4
    scratch = (
        [pltpu.VMEM((C,), jnp.int32)] * (4 * NSLOT) +
        [pltpu.VMEM((C, 16), jnp.float32)] * (nbuf * NSLOT) +
        [pltpu.VMEM((RPS, 16), jnp.float32)] +
        [pltpu.SemaphoreType.DMA] * (3 * NSLOT + 1) +
        [pltpu.VMEM_SHARED((NP, 16), jnp.float32)]
    )
    out_type = [jax.ShapeDtypeStruct((EP // C, C, 16), jnp.float32),
                jax.ShapeDtypeStruct((NCORES, NP, 16), jnp.float32)]
    if has_ltc:
        scratch = scratch + [pltpu.VMEM_SHARED((NP, 16), jnp.float32)]
        out_type.append(jax.ShapeDtypeStruct((NCORES, NP, 16), jnp.float32))

    def body(*refs):
        (src2d, dst2d, typ2d, as16, ad16, et16) = refs[:6]
        refs = refs[6:]
        if has_ltc:
            ltc16, ex_o, den_o, ltc_o = refs[:4]
            refs = refs[4:]
        else:
            ex_o, den_o = refs[:2]
            refs = refs[2:]
        sb = refs[:NSLOT]
        db = refs[NSLOT:2 * NSLOT]
        tb = refs[2 * NSLOT:3 * NSLOT]
        mdb = refs[3 * NSLOT:4 * NSLOT]
        k = 4 * NSLOT
        etb = refs[k:k + NSLOT]
        asb = refs[k + NSLOT:k + 2 * NSLOT]
        adb = refs[k + 2 * NSLOT:k + 3 * NSLOT]
        acc = refs[k + 3 * NSLOT:k + 4 * NSLOT]
        if has_ltc:
            ltb = refs[k + 4 * NSLOT:k + 5 * NSLOT]
        k = k + nbuf * NSLOT
        zb = refs[k]
        lsem = refs[k + 1:k + 1 + NSLOT]
        gsem = refs[k + 1 + NSLOT:k + 1 + 2 * NSLOT]
        osem = refs[k + 1 + 2 * NSLOT:k + 1 + 3 * NSLOT]
        zsem = refs[k + 1 + 3 * NSLOT]
        den_acc = refs[k + 2 + 3 * NSLOT]
        if has_ltc:
            ltc_acc = refs[k + 3 + 3 * NSLOT]
        c = lax.axis_index("c")
        s = lax.axis_index("s")

        @pl.loop(0, RPS)
        def _zero(i):
            zb[i] = jnp.zeros((16,), jnp.float32)

        pltpu.sync_copy(zb, den_acc.at[pl.ds(s * RPS, RPS)])
        if has_ltc:
            pltpu.sync_copy(zb, ltc_acc.at[pl.ds(s * RPS, RPS)])
        plsc.subcore_barrier()

        t0 = (c * NSUB + s) * NCH

        def issue_lin(q, g):
            return [pltpu.async_copy(src2d.at[t0 + g], sb[q], lsem[q]),
                    pltpu.async_copy(dst2d.at[t0 + g], db[q], lsem[q]),
                    pltpu.async_copy(typ2d.at[t0 + g], tb[q], lsem[q])]

        def mask_and_gather(q, lin):
            for cp in lin:
                cp.wait()

            @pl.loop(0, C // 16)
            def _mask(j):
                t = tb[q][pl.ds(j * 16, 16)]
                d = db[q][pl.ds(j * 16, 16)]
                mdb[q][pl.ds(j * 16, 16)] = _mask_dst(mode, t, d)

            cps = [pltpu.async_copy(et16.at[tb[q]], etb[q], gsem[q]),
                   pltpu.async_copy(as16.at[sb[q]], asb[q], gsem[q]),
                   pltpu.async_copy(ad16.at[mdb[q]], adb[q], gsem[q])]
            if has_ltc:
                cps.append(pltpu.async_copy(ltc16.at[tb[q]], ltb[q], gsem[q]))
            return cps

        def compute_and_out(q, g, gath):
            for cp in gath:
                cp.wait()

            @pl.loop(0, C)
            def _exp(e):
                v = etb[q][e] + asb[q][e] + adb[q][e]
                acc[q][e] = jnp.exp(jnp.where(v > 0, v, 0.2 * v))

            cps = [pltpu.async_copy(acc[q], ex_o.at[t0 + g], osem[q])]
            pltpu.sync_copy(acc[q], den_acc.at[mdb[q]], add=True)
            if has_ltc:
                pltpu.sync_copy(ltb[q], ltc_acc.at[mdb[q]], add=True)
            return cps

        @pl.loop(0, NCH // NSLOT)
        def _pair(kk):
            g0 = kk * NSLOT
            lins = [issue_lin(q, g0 + q) for q in range(NSLOT)]
            gaths = [mask_and_gather(q, lins[q]) for q in range(NSLOT)]
            outs = [compute_and_out(q, g0 + q, gaths[q]) for q in range(NSLOT)]
            for o in outs:
                for cp in o:
                    cp.wait()

        plsc.subcore_barrier()
        pltpu.sync_copy(den_acc.at[pl.ds(s * RPS, RPS)],
                        den_o.at[c, pl.ds(s * RPS, RPS)])
        if has_ltc:
            pltpu.sync_copy(ltc_acc.at[pl.ds(s * RPS, RPS)],
                            ltc_o.at[c, pl.ds(s * RPS, RPS)])

    return pl.kernel(body, out_type=tuple(out_type), mesh=_MESH,
                     scratch_types=tuple(scratch),
                     compiler_params=pltpu.CompilerParams(
                         use_tc_tiling_on_sc=False))


def _sc_pass_b(mode):
    NSLOT = 2
    scratch = (
        [pltpu.VMEM((C,), jnp.int32)] * (4 * NSLOT) +
        [pltpu.VMEM((C, 16), jnp.float32)] * (2 * NSLOT) +
        [pltpu.VMEM((C, D), jnp.float32)] * NSLOT +
        [pltpu.SemaphoreType.DMA] * (3 * NSLOT + 1) +
        [pltpu.VMEM_SHARED((NP, D), jnp.float32)]
    )

    def body(src2d, dst2d, typ2d, ex_i, rden, h, out_o, *refs):
        sb = refs[:NSLOT]
        db = refs[NSLOT:2 * NSLOT]
        tb = refs[2 * NSLOT:3 * NSLOT]
        mdb = refs[3 * NSLOT:4 * NSLOT]
        exb = refs[4 * NSLOT:5 * NSLOT]
        rdb = refs[5 * NSLOT:6 * NSLOT]
        hb = refs[6 * NSLOT:7 * NSLOT]
        k = 7 * NSLOT
        lsem = refs[k:k + NSLOT]
        gsem = refs[k + NSLOT:k + 2 * NSLOT]
        osem = refs[k + 2 * NSLOT:k + 3 * NSLOT]
        zsem = refs[k + 3 * NSLOT]
        out_acc = refs[k + 3 * NSLOT + 1]
        c = lax.axis_index("c")
        s = lax.axis_index("s")

        @pl.loop(0, C)
        def _zero(i):
            for j in range(D // 16):
                hb[0][i, pl.ds(j * 16, 16)] = jnp.zeros((16,), jnp.float32)

        for kk in range(RPS // C):
            pltpu.sync_copy(hb[0], out_acc.at[pl.ds(s * RPS + kk * C, C)])
        rem = RPS % C
        if rem:
            pltpu.sync_copy(
                hb[0].at[pl.ds(0, rem)],
                out_acc.at[pl.ds(s * RPS + (RPS // C) * C, rem)])
        plsc.subcore_barrier()

        t0 = (c * NSUB + s) * NCH

        def issue_lin(q, g):
            return [pltpu.async_copy(src2d.at[t0 + g], sb[q], lsem[q]),
                    pltpu.async_copy(dst2d.at[t0 + g], db[q], lsem[q]),
                    pltpu.async_copy(typ2d.at[t0 + g], tb[q], lsem[q]),
                    pltpu.async_copy(ex_i.at[t0 + g], exb[q], lsem[q])]

        def mask_and_gather(q, lin):
            for cp in lin:
                cp.wait()

            @pl.loop(0, C // 16)
            def _mask(j):
                t = tb[q][pl.ds(j * 16, 16)]
                d = db[q][pl.ds(j * 16, 16)]
                mdb[q][pl.ds(j * 16, 16)] = _mask_dst(mode, t, d)

            return [pltpu.async_copy(rden.at[mdb[q]], rdb[q], gsem[q]),
                    pltpu.async_copy(h.at[sb[q]], hb[q], gsem[q])]

        def compute_and_out(q, gath):
            for cp in gath:
                cp.wait()

            @pl.loop(0, C)
            def _scale(e):
                cfv = exb[q][e] * rdb[q][e]
                for j in range(D // 16):
                    hb[q][e, pl.ds(j * 16, 16)] = (
                        hb[q][e, pl.ds(j * 16, 16)] * cfv[j])

            pltpu.sync_copy(hb[q], out_acc.at[mdb[q]], add=True)

        @pl.loop(0, NCH // NSLOT)
        def _pair(kk):
            g0 = kk * NSLOT
            lins = [issue_lin(q, g0 + q) for q in range(NSLOT)]
            gaths = [mask_and_gather(q, lins[q]) for q in range(NSLOT)]
            for q in range(NSLOT):
                compute_and_out(q, gaths[q])

        plsc.subcore_barrier()
        pltpu.sync_copy(out_acc.at[pl.ds(s * RPS, RPS)],
                        out_o.at[c, pl.ds(s * RPS, RPS)])

    return pl.kernel(
        body,
        out_type=jax.ShapeDtypeStruct((NCORES, NP, D), jnp.float32),
        mesh=_MESH,
        scratch_types=tuple(scratch),
        compiler_params=pltpu.CompilerParams(use_tc_tiling_on_sc=False))


def _sel8():
    r = jnp.arange(D)[:, None]
    c_ = jnp.arange(16)[None, :]
    return jnp.where((r // 16) == c_, 1.0, 0.0).astype(jnp.float32)


def _expand16():
    r = jnp.arange(16)[:, None]
    c_ = jnp.arange(D)[None, :]
    return jnp.where(r == (c_ // 16), 1.0, 0.0).astype(jnp.float32)


def _pad_nodes(a):
    return jnp.pad(a, ((0, NP - N), (0, 0)))


def kernel(x, edge_index, batch, edge_attr, params):
    p1, p2, p3, p4 = (params['conv1'], params['conv2'],
                      params['conv3'], params['conv4'])
    sel8 = _sel8()
    sel1 = jnp.ones((D, 16), jnp.float32)
    expand = _expand16()

    pad = EP - E
    src0 = jnp.concatenate([edge_index[0].astype(jnp.int32),
                            jnp.zeros((pad,), jnp.int32)]).reshape(EP // C, C)
    dst0 = jnp.concatenate([edge_index[1].astype(jnp.int32),
                            jnp.zeros((pad,), jnp.int32)]).reshape(EP // C, C)
    typ = jnp.concatenate([edge_attr.astype(jnp.int32),
                           jnp.full((pad,), 7, jnp.int32)]).reshape(EP // C, C)

    we4 = jnp.stack([p['W_e'] for p in (p1, p2, p3, p4)])
    afe4 = jnp.stack([p['att_e'].reshape(1, -1) if p['att_e'].size == D
                      else p['att_e'].reshape(1, -1)
                      for p in (p1, p2, p3, p4)])
    sel4 = jnp.stack([sel8, sel8, sel1, sel1])
    oh = (jnp.arange(16) == 8).astype(jnp.float32).reshape(1, 16)
    et4, ltc2 = _tc_prep(params['edge_emb'], we4, afe4, sel4, oh)

    sc_a_loop = _sc_pass_a(0, True)
    sc_a3 = _sc_pass_a(1, False)
    sc_a4 = _sc_pass_a(2, False)
    sc_b = [_sc_pass_b(m) for m in (0, 0, 1, 2)]

    def gat_loop(li, hin, w, afs, afd, p, et16, ltc16, res, next_dense):
        h, a_s, a_d = hin
        ex, den_r, ltc_r = sc_a_loop(src0, dst0, typ,
                                     _pad_nodes(a_s), _pad_nodes(a_d),
                                     et16, ltc16)
        rden, cl = _tc_mid_loop(den_r[0, :N], den_r[1, :N],
                                ltc_r[0, :N], ltc_r[1, :N], a_s, a_d)
        outp = sc_b[li](src0, dst0, typ, ex, _pad_nodes(rden), h)
        b = p['b'].reshape(1, D)
        return _tc_ep(outp[0, :N], outp[1, :N], b,
                      (h, cl, expand), res, next_dense)

    def gat_plain(li, hin, p, sc_a, sarr, darr, res, next_dense):
        h, a_s, a_d = hin
        ex, den_r = sc_a(sarr, darr, typ,
                         _pad_nodes(a_s), _pad_nodes(a_d), et4[li])
        rden = _tc_mid_plain(den_r[0, :N], den_r[1, :N])
        outp = sc_b[li](sarr, darr, typ, ex, _pad_nodes(rden), h)
        b = p['b'].reshape(1, D)
        return _tc_ep(outp[0, :N], outp[1, :N], b, None, res, next_dense)

    def flat(p):
        return (p['W'], p['att_src'].reshape(1, D), p['att_dst'].reshape(1, D))

    w1, afs1, afd1 = flat(p1)
    w2, afs2, afd2 = flat(p2)
    w3, afs3, afd3 = flat(p3)
    w4, afs4, afd4 = flat(p4)

    h1, as1, ad1 = _tc_dense(x, w1, afs1, afd1, sel8)

    hA, h2f, as2, ad2 = gat_loop(0, (h1, as1, ad1), w1, afs1, afd1, p1,
                                 et4[0], ltc2[0], None,
                                 (w2, afs2, afd2, sel8))
    hB, h3f, as3, ad3 = gat_loop(1, (h2f, as2, ad2), w2, afs2, afd2, p2,
                                 et4[1], ltc2[1], hA,
                                 (w3, afs3, afd3, sel1))
    hC, h4f, as4, ad4 = gat_plain(2, (h3f, as3, ad3), p3, sc_a3,
                                  src0, dst0, hB,
                                  (w4, afs4, afd4, sel1))
    hD = gat_plain(3, (h4f, as4, ad4), p4, sc_a4,
                   dst0, src0, hC, None)
    return (hD, batch)


# speedup vs baseline: 32.9684x; 3.4833x over previous
import functools

import jax
import jax.numpy as jnp
from jax import lax
from jax.experimental import pallas as pl
from jax.experimental.pallas import tpu as pltpu
from jax.experimental.pallas import tpu_sc as plsc

N = 10000
D = 128
NP = 10112
E = 320000
NCORES = 2
NSUB = 16
NTILES = NCORES * NSUB
C = 128
EPT = 10240
EP = NTILES * EPT
NCH = EPT // C
RPS = NP // NSUB
ZR = 80
HD = 64

_MESH = plsc.VectorSubcoreMesh(core_axis_name="c", subcore_axis_name="s")


def _prep_body(ee_ref, we_ref, afe_ref, sel_ref, oh_ref, et_ref, ltc_ref):
    for i in range(4):
        ehh = jnp.dot(ee_ref[...], we_ref[i], preferred_element_type=jnp.float32)
        tab = jnp.dot(ehh * afe_ref[i], sel_ref[i],
                      preferred_element_type=jnp.float32)
        et_ref[i] = tab
        if i < 2:
            ltc_ref[i] = tab + oh_ref[...]


def _dense_body(x_ref, w_ref, afs_ref, afd_ref, sel_ref,
                h_ref, as_ref, ad_ref):
    h = jnp.dot(x_ref[...], w_ref[...], preferred_element_type=jnp.float32)
    h_ref[...] = h
    as_ref[...] = jnp.dot(h * afs_ref[...], sel_ref[...],
                          preferred_element_type=jnp.float32)
    ad_ref[...] = jnp.dot(h * afd_ref[...], sel_ref[...],
                          preferred_element_type=jnp.float32)


def _mid_loop_body(d0_ref, d1_ref, l0_ref, l1_ref, as_ref, ad_ref,
                   rden_ref, cl_ref):
    den = d0_ref[...] + d1_ref[...]
    ltc = l0_ref[...] + l1_ref[...]
    cnt = ltc[:, 8:9]
    la = as_ref[...] + ad_ref[...] + ltc / jnp.maximum(cnt, 1.0)
    lex = jnp.exp(jnp.where(la > 0, la, 0.2 * la))
    rden = 1.0 / jnp.maximum(den + lex, 1e-16)
    rden_ref[...] = rden
    cl_ref[...] = lex * rden


def _mid_plain_body(d0_ref, d1_ref, rden_ref):
    rden_ref[...] = 1.0 / jnp.maximum(d0_ref[...] + d1_ref[...], 1e-16)


def _ep_body(has_cl, has_res, has_dense, *refs):
    i = 0
    o0_ref = refs[i]; i += 1
    o1_ref = refs[i]; i += 1
    b_ref = refs[i]; i += 1
    g = o0_ref[...] + o1_ref[...] + b_ref[...]
    if has_cl:
        h_ref = refs[i]; i += 1
        cl_ref = refs[i]; i += 1
        ex_ref = refs[i]; i += 1
        g = g + jnp.dot(cl_ref[...], ex_ref[...],
                        preferred_element_type=jnp.float32) * h_ref[...]
    g = jnp.where(g > 0, g, 0.0)
    if has_res:
        r_ref = refs[i]; i += 1
        g = g + r_ref[...]
    if has_dense:
        w_ref = refs[i]; i += 1
        afs_ref = refs[i]; i += 1
        afd_ref = refs[i]; i += 1
        sel_ref = refs[i]; i += 1
        hn_ref = refs[i]; i += 1
        hf_ref = refs[i]; i += 1
        as_ref = refs[i]; i += 1
        ad_ref = refs[i]; i += 1
        hn_ref[...] = g
        hf = jnp.dot(g, w_ref[...], preferred_element_type=jnp.float32)
        hf_ref[...] = hf
        as_ref[...] = jnp.dot(hf * afs_ref[...], sel_ref[...],
                              preferred_element_type=jnp.float32)
        ad_ref[...] = jnp.dot(hf * afd_ref[...], sel_ref[...],
                              preferred_element_type=jnp.float32)
    else:
        hn_ref = refs[i]; i += 1
        hn_ref[...] = g


_GB = 10
_BN = N // _GB


def _row_spec(w):
    return pl.BlockSpec((_BN, w), lambda i: (i, 0))


def _full_spec(shape):
    return pl.BlockSpec(shape, lambda i: tuple(0 for _ in shape))


def _tc_dense(x, w, afs, afd, sel):
    return pl.pallas_call(
        _dense_body,
        grid=(_GB,),
        in_specs=[_row_spec(D), _full_spec((D, D)), _full_spec((1, D)),
                  _full_spec((1, D)), _full_spec((D, 16))],
        out_specs=[_row_spec(D), _row_spec(16), _row_spec(16)],
        out_shape=[jax.ShapeDtypeStruct((N, D), jnp.float32),
                   jax.ShapeDtypeStruct((N, 16), jnp.float32),
                   jax.ShapeDtypeStruct((N, 16), jnp.float32)],
    )(x, w, afs, afd, sel)


def _tc_mid_loop(d0, d1, l0, l1, a_s, a_d):
    return pl.pallas_call(
        _mid_loop_body,
        grid=(_GB,),
        in_specs=[_row_spec(16)] * 6,
        out_specs=[_row_spec(16)] * 2,
        out_shape=[jax.ShapeDtypeStruct((N, 16), jnp.float32),
                   jax.ShapeDtypeStruct((N, 16), jnp.float32)],
    )(d0, d1, l0, l1, a_s, a_d)


def _tc_mid_plain(d0, d1):
    return pl.pallas_call(
        _mid_plain_body,
        grid=(_GB,),
        in_specs=[_row_spec(16)] * 2,
        out_specs=_row_spec(16),
        out_shape=jax.ShapeDtypeStruct((N, 16), jnp.float32),
    )(d0, d1)


def _tc_ep(o0, o1, b, cl_args, res, dense_args):
    has_cl = cl_args is not None
    has_res = res is not None
    has_dense = dense_args is not None
    ins = [o0, o1, b]
    in_specs = [_row_spec(D), _row_spec(D), _full_spec((1, D))]
    if has_cl:
        hprev, cl, ex = cl_args
        ins += [hprev, cl, ex]
        in_specs += [_row_spec(D), _row_spec(16), _full_spec((16, D))]
    if has_res:
        ins.append(res)
        in_specs.append(_row_spec(D))
    if has_dense:
        w, afs, afd, sel = dense_args
        ins += [w, afs, afd, sel]
        in_specs += [_full_spec((D, D)), _full_spec((1, D)),
                     _full_spec((1, D)), _full_spec((D, 16))]
        out_specs = [_row_spec(D), _row_spec(D), _row_spec(16), _row_spec(16)]
        out_shape = [jax.ShapeDtypeStruct((N, D), jnp.float32),
                     jax.ShapeDtypeStruct((N, D), jnp.float32),
                     jax.ShapeDtypeStruct((N, 16), jnp.float32),
                     jax.ShapeDtypeStruct((N, 16), jnp.float32)]
    else:
        out_specs = _row_spec(D)
        out_shape = jax.ShapeDtypeStruct((N, D), jnp.float32)
    return pl.pallas_call(
        functools.partial(_ep_body, has_cl, has_res, has_dense),
        grid=(_GB,),
        in_specs=in_specs,
        out_specs=out_specs,
        out_shape=out_shape,
    )(*ins)


def _tc_prep(ee, we4, afe4, sel4, oh):
    return pl.pallas_call(
        _prep_body,
        out_shape=[jax.ShapeDtypeStruct((4, 8, 16), jnp.float32),
                   jax.ShapeDtypeStruct((2, 8, 16), jnp.float32)],
    )(ee, we4, afe4, sel4, oh)


def _mask_dst(mode, t, d):
    if mode == 0:
        m = t <= 1
    elif mode == 1:
        m = t == 2
    else:
        m = t == 1
    return jnp.where(m, d, N)


def _sc_pass_a(mode, has_ltc):
    NSLOT = 2
    nbuf = 5 if has_ltc else 4
    scratch = (
        [pltpu.VMEM((C,), jnp.int32)] * (4 * NSLOT) +
        [pltpu.VMEM((C, 16), jnp.float32)] * (nbuf * NSLOT) +
        [pltpu.VMEM((RPS, 16), jnp.float32)] +
        [pltpu.SemaphoreType.DMA] * (3 * NSLOT + 1) +
        [pltpu.VMEM_SHARED((NP, 16), jnp.float32)]
    )
    out_type = [jax.ShapeDtypeStruct((EP // C, C, 16), jnp.float32),
                jax.ShapeDtypeStruct((NCORES, NP, 16), jnp.float32)]
    if has_ltc:
        scratch = scratch + [pltpu.VMEM_SHARED((NP, 16), jnp.float32)]
        out_type.append(jax.ShapeDtypeStruct((NCORES, NP, 16), jnp.float32))
    scratch = scratch + [pltpu.VMEM_SHARED((NP, 16), jnp.float32),
                         pltpu.VMEM_SHARED((NP, 16), jnp.float32),
                         pltpu.VMEM_SHARED((8, 16), jnp.float32),
                         pltpu.VMEM_SHARED((8, 16), jnp.float32)]

    def body(*refs):
        (src2d, dst2d, typ2d, as16, ad16, et16) = refs[:6]
        refs = refs[6:]
        if has_ltc:
            ltc16, ex_o, den_o, ltc_o = refs[:4]
            refs = refs[4:]
        else:
            ex_o, den_o = refs[:2]
            refs = refs[2:]
        sb = refs[:NSLOT]
        db = refs[NSLOT:2 * NSLOT]
        tb = refs[2 * NSLOT:3 * NSLOT]
        mdb = refs[3 * NSLOT:4 * NSLOT]
        k = 4 * NSLOT
        etb = refs[k:k + NSLOT]
        asb = refs[k + NSLOT:k + 2 * NSLOT]
        adb = refs[k + 2 * NSLOT:k + 3 * NSLOT]
        acc = refs[k + 3 * NSLOT:k + 4 * NSLOT]
        if has_ltc:
            ltb = refs[k + 4 * NSLOT:k + 5 * NSLOT]
        k = k + nbuf * NSLOT
        zb = refs[k]
        lsem = refs[k + 1:k + 1 + NSLOT]
        gsem = refs[k + 1 + NSLOT:k + 1 + 2 * NSLOT]
        osem = refs[k + 1 + 2 * NSLOT:k + 1 + 3 * NSLOT]
        zsem = refs[k + 1 + 3 * NSLOT]
        den_acc = refs[k + 2 + 3 * NSLOT]
        j = k + 3 + 3 * NSLOT
        if has_ltc:
            ltc_acc = refs[j]
            j += 1
        as_s, ad_s, et_s, ltc_s = refs[j:j + 4]
        c = lax.axis_index("c")
        s = lax.axis_index("s")

        @pl.loop(0, RPS)
        def _zero(i):
            zb[i] = jnp.zeros((16,), jnp.float32)

        pltpu.sync_copy(zb, den_acc.at[pl.ds(s * RPS, RPS)])
        if has_ltc:
            pltpu.sync_copy(zb, ltc_acc.at[pl.ds(s * RPS, RPS)])
        pltpu.sync_copy(as16.at[pl.ds(s * RPS, RPS)],
                        as_s.at[pl.ds(s * RPS, RPS)])
        pltpu.sync_copy(ad16.at[pl.ds(s * RPS, RPS)],
                        ad_s.at[pl.ds(s * RPS, RPS)])

        @pl.when(s == 0)
        def _stage_small():
            pltpu.sync_copy(et16, et_s)
            if has_ltc:
                pltpu.sync_copy(ltc16, ltc_s)

        plsc.subcore_barrier()

        t0 = (c * NSUB + s) * NCH

        def issue_lin(q, g):
            return [pltpu.async_copy(src2d.at[t0 + g], sb[q], lsem[q]),
                    pltpu.async_copy(dst2d.at[t0 + g], db[q], lsem[q]),
                    pltpu.async_copy(typ2d.at[t0 + g], tb[q], lsem[q])]

        def mask_and_gather(q, lin):
            for cp in lin:
                cp.wait()

            @pl.loop(0, C // 16)
            def _mask(j):
                t = tb[q][pl.ds(j * 16, 16)]
                d = db[q][pl.ds(j * 16, 16)]
                mdb[q][pl.ds(j * 16, 16)] = _mask_dst(mode, t, d)

            cps = [pltpu.async_copy(et_s.at[tb[q]], etb[q], gsem[q]),
                   pltpu.async_copy(as_s.at[sb[q]], asb[q], gsem[q]),
                   pltpu.async_copy(ad_s.at[mdb[q]], adb[q], gsem[q])]
            if has_ltc:
                cps.append(pltpu.async_copy(ltc_s.at[tb[q]], ltb[q], gsem[q]))
            return cps

        def compute_and_out(q, g, gath):
            for cp in gath:
                cp.wait()

            @pl.loop(0, C)
            def _exp(e):
                v = etb[q][e] + asb[q][e] + adb[q][e]
                acc[q][e] = jnp.exp(jnp.where(v > 0, v, 0.2 * v))

            cps = [pltpu.async_copy(acc[q], ex_o.at[t0 + g], osem[q])]
            pltpu.sync_copy(acc[q], den_acc.at[mdb[q]], add=True)
            if has_ltc:
                pltpu.sync_copy(ltb[q], ltc_acc.at[mdb[q]], add=True)
            return cps

        @pl.loop(0, NCH // NSLOT)
        def _pair(kk):
            g0 = kk * NSLOT
            lins = [issue_lin(q, g0 + q) for q in range(NSLOT)]
            gaths = [mask_and_gather(q, lins[q]) for q in range(NSLOT)]
            outs = [compute_and_out(q, g0 + q, gaths[q]) for q in range(NSLOT)]
            for o in outs:
                for cp in o:
                    cp.wait()

        plsc.subcore_barrier()
        pltpu.sync_copy(den_acc.at[pl.ds(s * RPS, RPS)],
                        den_o.at[c, pl.ds(s * RPS, RPS)])
        if has_ltc:
            pltpu.sync_copy(ltc_acc.at[pl.ds(s * RPS, RPS)],
                            ltc_o.at[c, pl.ds(s * RPS, RPS)])

    return pl.kernel(body, out_type=tuple(out_type), mesh=_MESH,
                     scratch_types=tuple(scratch),
                     compiler_params=pltpu.CompilerParams(
                         use_tc_tiling_on_sc=False))


def _sc_pass_b(mode, hf):
    NSLOT = 2
    scratch = (
        [pltpu.VMEM((C,), jnp.int32)] * (4 * NSLOT) +
        [pltpu.VMEM((C, 16), jnp.float32)] * (2 * NSLOT) +
        [pltpu.VMEM((C, HD), jnp.float32)] * NSLOT +
        [pltpu.SemaphoreType.DMA] * (3 * NSLOT + 1) +
        [pltpu.VMEM_SHARED((NP, HD), jnp.float32)] +
        [pltpu.VMEM_SHARED((NP, HD), jnp.float32)] +
        [pltpu.VMEM_SHARED((NP, 16), jnp.float32)]
    )

    def body(src2d, dst2d, typ2d, ex_i, rden, h, out_o, *refs):
        sb = refs[:NSLOT]
        db = refs[NSLOT:2 * NSLOT]
        tb = refs[2 * NSLOT:3 * NSLOT]
        mdb = refs[3 * NSLOT:4 * NSLOT]
        exb = refs[4 * NSLOT:5 * NSLOT]
        rdb = refs[5 * NSLOT:6 * NSLOT]
        hb = refs[6 * NSLOT:7 * NSLOT]
        k = 7 * NSLOT
        lsem = refs[k:k + NSLOT]
        gsem = refs[k + NSLOT:k + 2 * NSLOT]
        osem = refs[k + 2 * NSLOT:k + 3 * NSLOT]
        zsem = refs[k + 3 * NSLOT]
        out_acc = refs[k + 3 * NSLOT + 1]
        h_s = refs[k + 3 * NSLOT + 2]
        rden_s = refs[k + 3 * NSLOT + 3]
        c = lax.axis_index("c")
        s = lax.axis_index("s")

        @pl.loop(0, C)
        def _zero(i):
            for j in range(HD // 16):
                hb[0][i, pl.ds(j * 16, 16)] = jnp.zeros((16,), jnp.float32)

        for kk in range(RPS // C):
            pltpu.sync_copy(hb[0], out_acc.at[pl.ds(s * RPS + kk * C, C)])
        rem = RPS % C
        if rem:
            pltpu.sync_copy(
                hb[0].at[pl.ds(0, rem)],
                out_acc.at[pl.ds(s * RPS + (RPS // C) * C, rem)])
        pltpu.sync_copy(h.at[pl.ds(s * RPS, RPS)], h_s.at[pl.ds(s * RPS, RPS)])
        pltpu.sync_copy(rden.at[pl.ds(s * RPS, RPS)],
                        rden_s.at[pl.ds(s * RPS, RPS)])
        plsc.subcore_barrier()

        t0 = (c * NSUB + s) * NCH

        def issue_lin(q, g):
            return [pltpu.async_copy(src2d.at[t0 + g], sb[q], lsem[q]),
                    pltpu.async_copy(dst2d.at[t0 + g], db[q], lsem[q]),
                    pltpu.async_copy(typ2d.at[t0 + g], tb[q], lsem[q]),
                    pltpu.async_copy(ex_i.at[t0 + g], exb[q], lsem[q])]

        def mask_and_gather(q, lin):
            for cp in lin:
                cp.wait()

            @pl.loop(0, C // 16)
            def _mask(j):
                t = tb[q][pl.ds(j * 16, 16)]
                d = db[q][pl.ds(j * 16, 16)]
                mdb[q][pl.ds(j * 16, 16)] = _mask_dst(mode, t, d)

            return [pltpu.async_copy(rden_s.at[mdb[q]], rdb[q], gsem[q]),
                    pltpu.async_copy(h_s.at[sb[q]], hb[q], gsem[q])]

        def compute_and_out(q, gath):
            for cp in gath:
                cp.wait()

            @pl.loop(0, C)
            def _scale(e):
                cfv = exb[q][e] * rdb[q][e]
                for j in range(HD // 16):
                    hb[q][e, pl.ds(j * 16, 16)] = (
                        hb[q][e, pl.ds(j * 16, 16)] * cfv[hf * (HD // 16) + j])

            pltpu.sync_copy(hb[q], out_acc.at[mdb[q]], add=True)

        @pl.loop(0, NCH // NSLOT)
        def _pair(kk):
            g0 = kk * NSLOT
            lins = [issue_lin(q, g0 + q) for q in range(NSLOT)]
            gaths = [mask_and_gather(q, lins[q]) for q in range(NSLOT)]
            for q in range(NSLOT):
                compute_and_out(q, gaths[q])

        plsc.subcore_barrier()
        pltpu.sync_copy(out_acc.at[pl.ds(s * RPS, RPS)],
                        out_o.at[c, pl.ds(s * RPS, RPS)])

    return pl.kernel(
        body,
        out_type=jax.ShapeDtypeStruct((NCORES, NP, HD), jnp.float32),
        mesh=_MESH,
        scratch_types=tuple(scratch),
        compiler_params=pltpu.CompilerParams(use_tc_tiling_on_sc=False))


def _sel8():
    r = jnp.arange(D)[:, None]
    c_ = jnp.arange(16)[None, :]
    return jnp.where((r // 16) == c_, 1.0, 0.0).astype(jnp.float32)


def _expand16():
    r = jnp.arange(16)[:, None]
    c_ = jnp.arange(D)[None, :]
    return jnp.where(r == (c_ // 16), 1.0, 0.0).astype(jnp.float32)


def _pad_nodes(a):
    return jnp.pad(a, ((0, NP - N), (0, 0)))


def kernel(x, edge_index, batch, edge_attr, params):
    p1, p2, p3, p4 = (params['conv1'], params['conv2'],
                      params['conv3'], params['conv4'])
    sel8 = _sel8()
    sel1 = jnp.ones((D, 16), jnp.float32)
    expand = _expand16()

    pad = EP - E
    src0 = jnp.concatenate([edge_index[0].astype(jnp.int32),
                            jnp.zeros((pad,), jnp.int32)]).reshape(EP // C, C)
    dst0 = jnp.concatenate([edge_index[1].astype(jnp.int32),
                            jnp.zeros((pad,), jnp.int32)]).reshape(EP // C, C)
    typ = jnp.concatenate([edge_attr.astype(jnp.int32),
                           jnp.full((pad,), 7, jnp.int32)]).reshape(EP // C, C)

    we4 = jnp.stack([p['W_e'] for p in (p1, p2, p3, p4)])
    afe4 = jnp.stack([p['att_e'].reshape(1, -1) if p['att_e'].size == D
                      else p['att_e'].reshape(1, -1)
                      for p in (p1, p2, p3, p4)])
    sel4 = jnp.stack([sel8, sel8, sel1, sel1])
    oh = (jnp.arange(16) == 8).astype(jnp.float32).reshape(1, 16)
    et4, ltc2 = _tc_prep(params['edge_emb'], we4, afe4, sel4, oh)

    sc_a_loop = _sc_pass_a(0, True)
    sc_a3 = _sc_pass_a(1, False)
    sc_a4 = _sc_pass_a(2, False)
    sc_b = [[_sc_pass_b(m, hf) for hf in (0, 1)] for m in (0, 0, 1, 2)]

    def gat_loop(li, hin, w, afs, afd, p, et16, ltc16, res, next_dense):
        h, a_s, a_d = hin
        ex, den_r, ltc_r = sc_a_loop(src0, dst0, typ,
                                     _pad_nodes(a_s), _pad_nodes(a_d),
                                     et16, ltc16)
        rden, cl = _tc_mid_loop(den_r[0, :N], den_r[1, :N],
                                ltc_r[0, :N], ltc_r[1, :N], a_s, a_d)
        rdp = _pad_nodes(rden)
        hp = _pad_nodes(h)
        oh0 = sc_b[li][0](src0, dst0, typ, ex, rdp, hp[:, :HD])
        oh1 = sc_b[li][1](src0, dst0, typ, ex, rdp, hp[:, HD:])
        o0 = jnp.concatenate([oh0[0, :N], oh1[0, :N]], axis=1)
        o1 = jnp.concatenate([oh0[1, :N], oh1[1, :N]], axis=1)
        b = p['b'].reshape(1, D)
        return _tc_ep(o0, o1, b, (h, cl, expand), res, next_dense)

    def gat_plain(li, hin, p, sc_a, sarr, darr, res, next_dense):
        h, a_s, a_d = hin
        ex, den_r = sc_a(sarr, darr, typ,
                         _pad_nodes(a_s), _pad_nodes(a_d), et4[li])
        rden = _tc_mid_plain(den_r[0, :N], den_r[1, :N])
        rdp = _pad_nodes(rden)
        hp = _pad_nodes(h)
        oh0 = sc_b[li][0](sarr, darr, typ, ex, rdp, hp[:, :HD])
        oh1 = sc_b[li][1](sarr, darr, typ, ex, rdp, hp[:, HD:])
        o0 = jnp.concatenate([oh0[0, :N], oh1[0, :N]], axis=1)
        o1 = jnp.concatenate([oh0[1, :N], oh1[1, :N]], axis=1)
        b = p['b'].reshape(1, D)
        return _tc_ep(o0, o1, b, None, res, next_dense)

    def flat(p):
        return (p['W'], p['att_src'].reshape(1, D), p['att_dst'].reshape(1, D))

    w1, afs1, afd1 = flat(p1)
    w2, afs2, afd2 = flat(p2)
    w3, afs3, afd3 = flat(p3)
    w4, afs4, afd4 = flat(p4)

    h1, as1, ad1 = _tc_dense(x, w1, afs1, afd1, sel8)

    hA, h2f, as2, ad2 = gat_loop(0, (h1, as1, ad1), w1, afs1, afd1, p1,
                                 et4[0], ltc2[0], None,
                                 (w2, afs2, afd2, sel8))
    hB, h3f, as3, ad3 = gat_loop(1, (h2f, as2, ad2), w2, afs2, afd2, p2,
                                 et4[1], ltc2[1], hA,
                                 (w3, afs3, afd3, sel1))
    hC, h4f, as4, ad4 = gat_plain(2, (h3f, as3, ad3), p3, sc_a3,
                                  src0, dst0, hB,
                                  (w4, afs4, afd4, sel1))
    hD = gat_plain(3, (h4f, as4, ad4), p4, sc_a4,
                   dst0, src0, hC, None)
    return (hD, batch)
